# baseline (speedup 1.0000x reference)
import functools

import jax
import jax.numpy as jnp
from jax import lax
from jax.experimental import pallas as pl
from jax.experimental.pallas import tpu as pltpu
from jax.experimental.pallas import tpu_sc as plsc

N_NODES = 10000
D_FEAT = 128
N_EDGES = 320000

NC, NS, L = 2, 16, 16
NW = NC * NS
EW = N_EDGES // NW
B = 80
NB = EW // B
G = B // L
C = D_FEAT // L
NBUF = 3

_mesh = plsc.VectorSubcoreMesh(core_axis_name="c", subcore_axis_name="s")

_DNUMS = lax.GatherDimensionNumbers(
    offset_dims=(), collapsed_slice_dims=(0,), start_index_map=(0,))


def _take16(x, idx2d):
    return lax.gather(x, idx2d, _DNUMS, slice_sizes=(1,),
                      mode=lax.GatherScatterMode.PROMISE_IN_BOUNDS)


_SCRATCH = (
    [pltpu.VMEM((EW,), jnp.int32)] * 2
    + [pltpu.VMEM((B, D_FEAT), jnp.float32)] * (2 * NBUF)
    + [pltpu.VMEM((EW,), jnp.float32)]
    + [pltpu.SemaphoreType.DMA] * (2 * NBUF)
)


@functools.partial(
    pl.kernel,
    out_type=jax.ShapeDtypeStruct((N_EDGES,), jnp.float32),
    mesh=_mesh,
    scratch_types=_SCRATCH,
)
def _gae_decode(z_hbm, ei_hbm, out_hbm, idx_s, idx_d, *rest):
    rows = rest[:2 * NBUF]
    out_v = rest[2 * NBUF]
    sems = rest[2 * NBUF + 1:]
    bufs = tuple((rows[2 * i], rows[2 * i + 1], sems[2 * i], sems[2 * i + 1])
                 for i in range(NBUF))

    wid = lax.axis_index("s") * NC + lax.axis_index("c")
    base = wid * EW
    lane = lax.iota(jnp.int32, L)
    perms = {sh: (lane ^ sh)[:, None] for sh in (1, 2, 4, 8)}
    mask1 = (lane & 1) == 0
    pair_masks = [(lane >> 1) == k for k in range(8)]

    pltpu.sync_copy(ei_hbm.at[pl.ds(base, EW)], idx_s)
    pltpu.sync_copy(ei_hbm.at[pl.ds(N_EDGES + base, EW)], idx_d)

    def fire(g, slot):
        rs, rd, ss, sd = bufs[slot]
        pltpu.async_copy(z_hbm.at[idx_s.at[pl.ds(g * B, B)]], rs, ss)
        pltpu.async_copy(z_hbm.at[idx_d.at[pl.ds(g * B, B)]], rd, sd)

    def consume(g, slot):
        rs, rd, ss, sd = bufs[slot]
        pltpu.make_async_copy(z_hbm.at[idx_s.at[pl.ds(0, B)]], rs, ss).wait()
        pltpu.make_async_copy(z_hbm.at[idx_d.at[pl.ds(0, B)]], rd, sd).wait()

        def grp_body(grp, _):
            row0 = grp * L
            out_vec = jnp.zeros((L,), jnp.float32)
            prev = None
            for e in range(L):
                row = row0 + e
                acc = rs[row, pl.ds(0, L)] * rd[row, pl.ds(0, L)]
                acc2 = rs[row, pl.ds(L, L)] * rd[row, pl.ds(L, L)]
                for c in range(2, C, 2):
                    acc += rs[row, pl.ds(c * L, L)] * rd[row, pl.ds(c * L, L)]
                    acc2 += (rs[row, pl.ds((c + 1) * L, L)]
                             * rd[row, pl.ds((c + 1) * L, L)])
                acc = acc + acc2
                if e % 2 == 0:
                    prev = acc
                    continue
                v = jnp.where(mask1, prev + _take16(prev, perms[1]),
                              acc + _take16(acc, perms[1]))
                for sh in (2, 4, 8):
                    v = v + _take16(v, perms[sh])
                out_vec = jnp.where(pair_masks[e // 2], v, out_vec)
            sig = 1.0 / (1.0 + jnp.exp(-out_vec))
            out_v[pl.ds(g * B + grp * L, L)] = sig
            return 0

        lax.fori_loop(0, G, grp_body, 0)

    for s in range(NBUF):
        fire(s, s)

    def body(k, _):
        g = k * NBUF
        for s in range(NBUF):
            consume(g + s, s)

            @pl.when(g + s + NBUF < NB)
            def _():
                fire(g + s + NBUF, s)

        return 0

    lax.fori_loop(0, NB // NBUF, body, 0)
    for t in range(NB % NBUF):
        consume(NB - (NB % NBUF) + t, t)

    pltpu.sync_copy(out_v, out_hbm.at[pl.ds(base, EW)])


def _pack_bf16_pairs(z):
    u = jax.lax.bitcast_convert_type(z, jnp.uint32)
    h = (u + 0x7FFF + ((u >> 16) & 1)) >> 16
    packed = h[:, 0::2] | (h[:, 1::2] << 16)
    return packed.astype(jnp.int32)


def kernel(z, edge_index):
    return _gae_decode(z, edge_index.astype(jnp.int32).reshape(-1))


# speedup vs baseline: 1.1582x; 1.1582x over previous
import functools

import jax
import jax.numpy as jnp
from jax import lax
from jax.experimental import pallas as pl
from jax.experimental.pallas import tpu as pltpu
from jax.experimental.pallas import tpu_sc as plsc

N_NODES = 10000
D_FEAT = 128
N_EDGES = 320000

NC, NS, L = 2, 16, 16
NW = NC * NS
EW = N_EDGES // NW
B = 80
NB = EW // B
G = B // L
C = D_FEAT // L
NBUF = 3

_mesh = plsc.VectorSubcoreMesh(core_axis_name="c", subcore_axis_name="s")

_DNUMS = lax.GatherDimensionNumbers(
    offset_dims=(), collapsed_slice_dims=(0,), start_index_map=(0,))


def _take16(x, idx2d):
    return lax.gather(x, idx2d, _DNUMS, slice_sizes=(1,),
                      mode=lax.GatherScatterMode.PROMISE_IN_BOUNDS)


_SCRATCH = (
    [pltpu.VMEM((EW,), jnp.int32)] * 2
    + [pltpu.VMEM((B, D_FEAT), jnp.float32)] * (2 * NBUF)
    + [pltpu.VMEM((EW,), jnp.float32)]
    + [pltpu.SemaphoreType.DMA] * (2 * NBUF)
)


@functools.partial(
    pl.kernel,
    out_type=jax.ShapeDtypeStruct((N_EDGES,), jnp.float32),
    mesh=_mesh,
    scratch_types=_SCRATCH,
)
def _gae_decode(z_hbm, ei_hbm, out_hbm, idx_s, idx_d, *rest):
    rows = rest[:2 * NBUF]
    out_v = rest[2 * NBUF]
    sems = rest[2 * NBUF + 1:]
    bufs = tuple((rows[2 * i], rows[2 * i + 1], sems[2 * i], sems[2 * i + 1])
                 for i in range(NBUF))

    wid = lax.axis_index("s") * NC + lax.axis_index("c")
    base = wid * EW
    lane = lax.iota(jnp.int32, L)
    perms = [(lane ^ sh)[:, None] for sh in (8, 4, 2, 1)]

    pltpu.sync_copy(ei_hbm.at[pl.ds(base, EW)], idx_s)
    pltpu.sync_copy(ei_hbm.at[pl.ds(N_EDGES + base, EW)], idx_d)

    def fire(g, slot):
        rs, rd, ss, sd = bufs[slot]
        pltpu.async_copy(z_hbm.at[idx_s.at[pl.ds(g * B, B)]], rs, ss)
        pltpu.async_copy(z_hbm.at[idx_d.at[pl.ds(g * B, B)]], rd, sd)

    def consume(g, slot):
        rs, rd, ss, sd = bufs[slot]
        pltpu.make_async_copy(z_hbm.at[idx_s.at[pl.ds(0, B)]], rs, ss).wait()
        pltpu.make_async_copy(z_hbm.at[idx_d.at[pl.ds(0, B)]], rd, sd).wait()

        def grp_body(grp, _):
            row0 = grp * L
            out_vec = jnp.zeros((L,), jnp.float32)
            for e in range(L):
                row = row0 + e
                acc = rs[row, pl.ds(0, L)] * rd[row, pl.ds(0, L)]
                acc2 = rs[row, pl.ds(L, L)] * rd[row, pl.ds(L, L)]
                for c in range(2, C, 2):
                    acc += rs[row, pl.ds(c * L, L)] * rd[row, pl.ds(c * L, L)]
                    acc2 += (rs[row, pl.ds((c + 1) * L, L)]
                             * rd[row, pl.ds((c + 1) * L, L)])
                acc = acc + acc2
                for p in perms:
                    acc = acc + _take16(acc, p)
                out_vec = jnp.where(lane == e, acc, out_vec)
            sig = 1.0 / (1.0 + jnp.exp(-out_vec))
            out_v[pl.ds(g * B + grp * L, L)] = sig
            return 0

        lax.fori_loop(0, G, grp_body, 0)

    for s in range(NBUF):
        fire(s, s)

    def body(k, _):
        g = k * NBUF
        for s in range(NBUF):
            consume(g + s, s)

            @pl.when(g + s + NBUF < NB)
            def _():
                fire(g + s + NBUF, s)

        return 0

    lax.fori_loop(0, NB // NBUF, body, 0)
    for t in range(NB % NBUF):
        consume(NB - (NB % NBUF) + t, t)

    pltpu.sync_copy(out_v, out_hbm.at[pl.ds(base, EW)])


def kernel(z, edge_index):
    return _gae_decode(z, edge_index.astype(jnp.int32).reshape(-1))


# speedup vs baseline: 1.2214x; 1.0546x over previous
import functools

import jax
import jax.numpy as jnp
from jax import lax
from jax.experimental import pallas as pl
from jax.experimental.pallas import tpu as pltpu
from jax.experimental.pallas import tpu_sc as plsc

N_NODES = 10000
D_FEAT = 128
N_EDGES = 320000

NC, NS, L = 2, 16, 16
NW = NC * NS
EW = N_EDGES // NW
B = 80
NB = EW // B
G = B // L
C = D_FEAT // L
NBUF = 2

_mesh = plsc.VectorSubcoreMesh(core_axis_name="c", subcore_axis_name="s")

_DNUMS = lax.GatherDimensionNumbers(
    offset_dims=(), collapsed_slice_dims=(0,), start_index_map=(0,))


def _take16(x, idx2d):
    return lax.gather(x, idx2d, _DNUMS, slice_sizes=(1,),
                      mode=lax.GatherScatterMode.PROMISE_IN_BOUNDS)


_SCRATCH = (
    [pltpu.VMEM((EW,), jnp.int32)] * 2
    + [pltpu.VMEM((B, D_FEAT), jnp.float32)] * (2 * NBUF)
    + [pltpu.VMEM((EW,), jnp.float32)]
    + [pltpu.SemaphoreType.DMA] * (2 * NBUF)
)


@functools.partial(
    pl.kernel,
    out_type=jax.ShapeDtypeStruct((N_EDGES,), jnp.float32),
    mesh=_mesh,
    scratch_types=_SCRATCH,
)
def _gae_decode(z_hbm, ei_hbm, out_hbm, idx_s, idx_d, *rest):
    rows = rest[:2 * NBUF]
    out_v = rest[2 * NBUF]
    sems = rest[2 * NBUF + 1:]
    bufs = tuple((rows[2 * i], rows[2 * i + 1], sems[2 * i], sems[2 * i + 1])
                 for i in range(NBUF))

    wid = lax.axis_index("s") * NC + lax.axis_index("c")
    base = wid * EW
    lane = lax.iota(jnp.int32, L)
    perms = [(lane ^ sh)[:, None] for sh in (8, 4, 2, 1)]

    pltpu.sync_copy(ei_hbm.at[pl.ds(base, EW)], idx_s)
    pltpu.sync_copy(ei_hbm.at[pl.ds(N_EDGES + base, EW)], idx_d)

    def fire(g, slot):
        rs, rd, ss, sd = bufs[slot]
        pltpu.async_copy(z_hbm.at[idx_s.at[pl.ds(g * B, B)]], rs, ss)
        pltpu.async_copy(z_hbm.at[idx_d.at[pl.ds(g * B, B)]], rd, sd)

    def consume(g, slot):
        rs, rd, ss, sd = bufs[slot]
        pltpu.make_async_copy(z_hbm.at[idx_s.at[pl.ds(0, B)]], rs, ss).wait()
        pltpu.make_async_copy(z_hbm.at[idx_d.at[pl.ds(0, B)]], rd, sd).wait()

        def grp_body(grp, _):
            row0 = grp * L
            out_vec = jnp.zeros((L,), jnp.float32)
            for e in range(L):
                row = row0 + e
                acc = rs[row, pl.ds(0, L)] * rd[row, pl.ds(0, L)]
                acc2 = rs[row, pl.ds(L, L)] * rd[row, pl.ds(L, L)]
                for c in range(2, C, 2):
                    acc += rs[row, pl.ds(c * L, L)] * rd[row, pl.ds(c * L, L)]
                    acc2 += (rs[row, pl.ds((c + 1) * L, L)]
                             * rd[row, pl.ds((c + 1) * L, L)])
                acc = acc + acc2
                for p in perms:
                    acc = acc + _take16(acc, p)
                out_vec = jnp.where(lane == e, acc, out_vec)
            sig = 1.0 / (1.0 + jnp.exp(-out_vec))
            out_v[pl.ds(g * B + grp * L, L)] = sig
            return 0

        lax.fori_loop(0, G, grp_body, 0)

    for s in range(NBUF):
        fire(s, s)

    def body(k, _):
        g = k * NBUF
        for s in range(NBUF):
            consume(g + s, s)

            @pl.when(g + s + NBUF < NB)
            def _():
                fire(g + s + NBUF, s)

        return 0

    lax.fori_loop(0, NB // NBUF, body, 0)
    for t in range(NB % NBUF):
        consume(NB - (NB % NBUF) + t, t)

    pltpu.sync_copy(out_v, out_hbm.at[pl.ds(base, EW)])


def kernel(z, edge_index):
    return _gae_decode(z, edge_index.astype(jnp.int32).reshape(-1))
